# baseline (speedup 1.0000x reference)
import functools

import jax
import jax.numpy as jnp
from jax import lax
from jax.experimental import pallas as pl
from jax.experimental.pallas import tpu as pltpu
from jax.experimental.pallas import tpu_sc as plsc

Q_LEN = 2048
K_LEN = 2048
EMB = 64
NWORKERS = 32
PAIRS_PER_WORKER = Q_LEN // 2 // NWORKERS


def _sc_body(rev_hbm, out_hbm, rev_sh, sem_a, sem_b):
    s = lax.axis_index("s")
    c = lax.axis_index("c")
    wid = s * 2 + c

    @pl.when(s == 0)
    def _():
        pltpu.sync_copy(rev_hbm, rev_sh)

    plsc.subcore_barrier()

    def pair(p, carry):
        a = pltpu.make_async_copy(
            rev_sh.at[pl.ds(K_LEN - 1 - 2 * p, K_LEN), :],
            out_hbm.at[2 * p], sem_a)
        b = pltpu.make_async_copy(
            rev_sh.at[pl.ds(K_LEN - 2 - 2 * p, K_LEN), :],
            out_hbm.at[2 * p + 1], sem_b)
        a.start()
        b.start()
        a.wait()
        b.wait()
        return carry

    lax.fori_loop(wid * PAIRS_PER_WORKER, (wid + 1) * PAIRS_PER_WORKER,
                  pair, 0)


_sc_call = functools.partial(
    pl.kernel,
    out_type=jax.ShapeDtypeStruct((Q_LEN, K_LEN, EMB), jnp.float32),
    mesh=plsc.VectorSubcoreMesh(core_axis_name="c", subcore_axis_name="s"),
    scratch_types=[
        pltpu.VMEM_SHARED((2 * K_LEN, EMB), jnp.float32),
        pltpu.SemaphoreType.DMA,
        pltpu.SemaphoreType.DMA,
    ],
)(_sc_body)


def kernel(q, k, rel_emb):
    rev = jnp.flip(rel_emb, axis=0)
    return _sc_call(rev)


# speedup vs baseline: 5.4979x; 5.4979x over previous
import jax
import jax.numpy as jnp
from jax.experimental import pallas as pl
from jax.experimental.pallas import tpu as pltpu

Q_LEN = 2048
K_LEN = 2048
EMB = 64
BI = 8


def _body(trev_ref, out_ref):
    i0 = pl.program_id(0) * BI
    for r in range(BI):
        w = K_LEN - 1 - (i0 + r)
        base = pl.multiple_of((w // 128) * 128, 128)
        b = w - base
        t1 = trev_ref[:, pl.ds(base, K_LEN + 128)]
        rolled = pltpu.roll(t1, -b, axis=1)
        out_ref[r] = rolled[:, :K_LEN]


def kernel(q, k, rel_emb):
    trev = jnp.flip(rel_emb, axis=0).T
    out_t = pl.pallas_call(
        _body,
        grid=(Q_LEN // BI,),
        in_specs=[
            pl.BlockSpec((EMB, 2 * K_LEN), lambda g: (0, 0),
                         memory_space=pltpu.VMEM),
        ],
        out_specs=pl.BlockSpec((BI, EMB, K_LEN), lambda g: (g, 0, 0)),
        out_shape=jax.ShapeDtypeStruct((Q_LEN, EMB, K_LEN), jnp.float32),
    )(trev)
    return jnp.transpose(out_t, (0, 2, 1))
